# baseline (speedup 1.0000x reference)
import functools

import jax
import jax.numpy as jnp
from jax import lax
from jax.experimental import pallas as pl
from jax.experimental.pallas import tpu as pltpu
from jax.experimental.pallas import tpu_sc as plsc

N_NODES = 10000
N_EDGES = 160000
FEAT = 256
HALF = 128

NC = 2
NS = 16

EPAD = 163840
ROWS_PER_TILE = EPAD // NS // HALF
ACC_ROWS = 10112
ACC_PER_TILE = ACC_ROWS // NS


def _sc_gather_probe(f_cat, srcp):
  mesh = plsc.VectorSubcoreMesh(
      core_axis_name="c", subcore_axis_name="s", num_cores=NC, num_subcores=NS)

  @functools.partial(
      pl.kernel,
      out_type=jax.ShapeDtypeStruct((EPAD, HALF), jnp.float32),
      mesh=mesh,
      scratch_types=[
          pltpu.VMEM((HALF,), jnp.int32),
          pltpu.VMEM((HALF, HALF), jnp.float32),
          pltpu.SemaphoreType.DMA,
      ],
  )
  def k(f_hbm, src_hbm, out_hbm, idx_v, rows_v, sem):
    cid = lax.axis_index("c")
    sid = lax.axis_index("s")
    wid = sid * NC + cid
    nbatch = EPAD // HALF // (NC * NS)

    @pl.loop(0, nbatch)
    def _(j):
      e = (wid * nbatch + j) * HALF
      pltpu.sync_copy(src_hbm.at[pl.ds(e, HALF)], idx_v)
      pltpu.async_copy(f_hbm.at[idx_v], rows_v, sem).wait()
      pltpu.sync_copy(rows_v, out_hbm.at[pl.ds(e, HALF)])

  return k(f_cat, srcp)


def _acc_chunks():
  chunks = [(c * HALF, HALF) for c in range(ACC_PER_TILE // HALF)]
  tail = ACC_PER_TILE % HALF
  if tail:
    chunks.append(((ACC_PER_TILE // HALF) * HALF, tail))
  return chunks


def _sc_mesh():
  return plsc.VectorSubcoreMesh(
      core_axis_name="c", subcore_axis_name="s", num_cores=NC, num_subcores=NS)


def _sc_sum(f_cat, src2, dstr, zeros_acc):
  IH = ROWS_PER_TILE // 2

  @functools.partial(
      pl.kernel,
      out_type=jax.ShapeDtypeStruct((NC, ACC_ROWS, HALF), jnp.float32),
      mesh=_sc_mesh(),
      scratch_types=[
          pltpu.VMEM_SHARED((ACC_ROWS, HALF), jnp.float32),
          pltpu.VMEM((IH, HALF), jnp.int32),
          pltpu.VMEM((IH, HALF), jnp.int32),
          pltpu.VMEM((HALF, HALF), jnp.float32),
          pltpu.VMEM((HALF, HALF), jnp.float32),
          pltpu.SemaphoreType.DMA,
          pltpu.SemaphoreType.DMA,
          pltpu.SemaphoreType.DMA,
          pltpu.SemaphoreType.DMA,
      ],
  )
  def k(f_hbm, src_hbm, dst_hbm, zacc_hbm, sum_out, acc, src_v, dst_v,
        rows_a, rows_b, sem_a, sem_b, sems_a, sems_b):
    cid = lax.axis_index("c")
    sid = lax.axis_index("s")
    a0 = sid * ACC_PER_TILE

    pltpu.sync_copy(zacc_hbm, rows_a)
    for off, n in _acc_chunks():
      pltpu.sync_copy(rows_a.at[pl.ds(0, n)], acc.at[pl.ds(a0 + off, n)])

    plsc.subcore_barrier()

    @pl.loop(0, 2)
    def _(h):
      r0 = sid * ROWS_PER_TILE + h * IH
      pltpu.sync_copy(src_hbm.at[cid, pl.ds(r0, IH)], src_v)
      pltpu.sync_copy(dst_hbm.at[pl.ds(r0, IH)], dst_v)

      def fire_g(b, rows, sem):
        pltpu.async_copy(f_hbm.at[src_v.at[b]], rows, sem)

      def wait_g(b, rows, sem):
        pltpu.make_async_copy(f_hbm.at[src_v.at[b]], rows, sem).wait()

      def fire_s(b, rows, ssem):
        pltpu.async_copy(rows, acc.at[dst_v.at[b]], add=True, sem=ssem)

      def wait_s(b, rows, ssem):
        pltpu.make_async_copy(rows, acc.at[dst_v.at[b]], ssem).wait()

      fire_g(0, rows_a, sem_a)
      fire_g(1, rows_b, sem_b)

      @pl.loop(0, IH // 2 - 1)
      def _(j):
        wait_g(2 * j, rows_a, sem_a)
        fire_s(2 * j, rows_a, sems_a)
        wait_g(2 * j + 1, rows_b, sem_b)
        fire_s(2 * j + 1, rows_b, sems_b)
        wait_s(2 * j, rows_a, sems_a)
        fire_g(2 * j + 2, rows_a, sem_a)
        wait_s(2 * j + 1, rows_b, sems_b)
        fire_g(2 * j + 3, rows_b, sem_b)

      wait_g(IH - 2, rows_a, sem_a)
      fire_s(IH - 2, rows_a, sems_a)
      wait_g(IH - 1, rows_b, sem_b)
      fire_s(IH - 1, rows_b, sems_b)
      wait_s(IH - 2, rows_a, sems_a)
      wait_s(IH - 1, rows_b, sems_b)

    plsc.subcore_barrier()

    for off, n in _acc_chunks():
      pltpu.sync_copy(acc.at[pl.ds(a0 + off, n)], rows_a.at[pl.ds(0, n)])
      pltpu.sync_copy(rows_a.at[pl.ds(0, n)],
                      sum_out.at[cid, pl.ds(a0 + off, n)])

  return k(f_cat, src2, dstr, zeros_acc)


DEG_BATCHES = EPAD // NC // NS // HALF
DEG_ROWS = 10240
DEG_PER_TILE = DEG_ROWS // NS


def _sc_deg(dstr, zeros_rows, ones_rows):
  @functools.partial(
      pl.kernel,
      out_type=jax.ShapeDtypeStruct((NC, DEG_ROWS, HALF), jnp.float32),
      mesh=_sc_mesh(),
      scratch_types=[
          pltpu.VMEM_SHARED((DEG_ROWS, HALF), jnp.float32),
          pltpu.VMEM((DEG_BATCHES, HALF), jnp.int32),
          pltpu.VMEM((HALF, HALF), jnp.float32),
          pltpu.SemaphoreType.DMA,
      ],
  )
  def k(dst_hbm, zrows_hbm, ones_hbm, deg_out, deg, dst_v, ones_v, sem):
    cid = lax.axis_index("c")
    sid = lax.axis_index("s")
    a0 = sid * DEG_PER_TILE
    e0 = cid * NS * DEG_BATCHES + sid * DEG_BATCHES

    pltpu.sync_copy(zrows_hbm, ones_v)

    @pl.loop(0, DEG_PER_TILE // HALF)
    def _(i):
      pltpu.sync_copy(ones_v, deg.at[pl.ds(a0 + i * HALF, HALF)])

    pltpu.sync_copy(ones_hbm, ones_v)
    pltpu.sync_copy(dst_hbm.at[pl.ds(e0, DEG_BATCHES)], dst_v)

    plsc.subcore_barrier()

    @pl.loop(0, DEG_BATCHES)
    def _(j):
      pltpu.async_copy(ones_v, deg.at[dst_v.at[j]], add=True, sem=sem)

    @pl.loop(0, DEG_BATCHES)
    def _(j):
      pltpu.make_async_copy(ones_v, deg.at[dst_v.at[j]], sem).wait()

    plsc.subcore_barrier()

    @pl.loop(0, DEG_PER_TILE // HALF)
    def _(i):
      pltpu.sync_copy(deg.at[pl.ds(a0 + i * HALF, HALF)], ones_v)
      pltpu.sync_copy(ones_v, deg_out.at[cid, pl.ds(a0 + i * HALF, HALF)])

  return k(dstr, zeros_rows, ones_rows)


def _tc_mlp_body(feat_ref, lo_ref, hi_ref, dega_ref, degb_ref, w1_ref, b1_ref,
                 w2_ref, b2_ref, out_ref):
  deg = dega_ref[0][:, 0:1] + degb_ref[0][:, 0:1]
  dinv = 1.0 / jnp.maximum(deg, 1.0)
  lo = lo_ref[0] * dinv
  hi = hi_ref[0] * dinv
  x = feat_ref[...]
  pre = jnp.dot(x, w1_ref[0:FEAT], preferred_element_type=jnp.float32)
  pre += jnp.dot(lo, w1_ref[FEAT:FEAT + HALF], preferred_element_type=jnp.float32)
  pre += jnp.dot(hi, w1_ref[FEAT + HALF:2 * FEAT], preferred_element_type=jnp.float32)
  h = jnp.tanh(pre + b1_ref[...])
  out_ref[...] = (jnp.dot(h, w2_ref[...], preferred_element_type=jnp.float32)
                  + b2_ref[...])


def _tc_mlp(features, summed, deg16, W1, b1, W2, b2):
  R = 2000
  grid = (N_NODES // R,)
  return pl.pallas_call(
      _tc_mlp_body,
      grid=grid,
      in_specs=[
          pl.BlockSpec((R, FEAT), lambda i: (i, 0)),
          pl.BlockSpec((1, R, HALF), lambda i: (0, i, 0)),
          pl.BlockSpec((1, R, HALF), lambda i: (1, i, 0)),
          pl.BlockSpec((1, R, HALF), lambda i: (0, i, 0)),
          pl.BlockSpec((1, R, HALF), lambda i: (1, i, 0)),
          pl.BlockSpec((2 * FEAT, FEAT), lambda i: (0, 0)),
          pl.BlockSpec((1, FEAT), lambda i: (0, 0)),
          pl.BlockSpec((FEAT, FEAT), lambda i: (0, 0)),
          pl.BlockSpec((1, FEAT), lambda i: (0, 0)),
      ],
      out_specs=pl.BlockSpec((R, FEAT), lambda i: (i, 0)),
      out_shape=jax.ShapeDtypeStruct((N_NODES, FEAT), jnp.float32),
  )(features, summed, summed, deg16, deg16, W1, b1.reshape(1, FEAT), W2,
    b2.reshape(1, FEAT))


def kernel(nodes, features, edge_index, W1, b1, W2, b2):
  features = features.astype(jnp.float32)
  src = edge_index[0].astype(jnp.int32)
  dst = edge_index[1].astype(jnp.int32)

  f_cat = jnp.concatenate([features[:, :HALF], features[:, HALF:]], axis=0)

  pad = EPAD - N_EDGES
  srcp = jnp.concatenate([src, jnp.zeros((pad,), jnp.int32)])
  dstp = jnp.concatenate([dst, jnp.full((pad,), N_NODES, jnp.int32)])
  srcr = srcp.reshape(EPAD // HALF, HALF)
  src2 = jnp.stack([srcr, srcr + N_NODES])
  dstr = dstp.reshape(EPAD // HALF, HALF)

  zeros_acc = jnp.zeros((HALF, HALF), jnp.float32)
  ones_rows = jnp.ones((HALF, HALF), jnp.float32)

  summed = _sc_sum(f_cat, src2, dstr, zeros_acc)
  deg16 = _sc_deg(dstr, zeros_acc, ones_rows)
  return _tc_mlp(features, summed, deg16, W1, b1, W2, b2)


# speedup vs baseline: 1.0654x; 1.0654x over previous
import functools

import jax
import jax.numpy as jnp
from jax import lax
from jax.experimental import pallas as pl
from jax.experimental.pallas import tpu as pltpu
from jax.experimental.pallas import tpu_sc as plsc

N_NODES = 10000
N_EDGES = 160000
FEAT = 256
HALF = 128

NC = 2
NS = 16

EPAD = 163840
ROWS_PER_TILE = EPAD // NS // HALF
ACC_ROWS = 10112
ACC_PER_TILE = ACC_ROWS // NS


def _sc_gather_probe(f_cat, srcp):
  mesh = plsc.VectorSubcoreMesh(
      core_axis_name="c", subcore_axis_name="s", num_cores=NC, num_subcores=NS)

  @functools.partial(
      pl.kernel,
      out_type=jax.ShapeDtypeStruct((EPAD, HALF), jnp.float32),
      mesh=mesh,
      scratch_types=[
          pltpu.VMEM((HALF,), jnp.int32),
          pltpu.VMEM((HALF, HALF), jnp.float32),
          pltpu.SemaphoreType.DMA,
      ],
  )
  def k(f_hbm, src_hbm, out_hbm, idx_v, rows_v, sem):
    cid = lax.axis_index("c")
    sid = lax.axis_index("s")
    wid = sid * NC + cid
    nbatch = EPAD // HALF // (NC * NS)

    @pl.loop(0, nbatch)
    def _(j):
      e = (wid * nbatch + j) * HALF
      pltpu.sync_copy(src_hbm.at[pl.ds(e, HALF)], idx_v)
      pltpu.async_copy(f_hbm.at[idx_v], rows_v, sem).wait()
      pltpu.sync_copy(rows_v, out_hbm.at[pl.ds(e, HALF)])

  return k(f_cat, srcp)


def _acc_chunks():
  chunks = [(c * HALF, HALF) for c in range(ACC_PER_TILE // HALF)]
  tail = ACC_PER_TILE % HALF
  if tail:
    chunks.append(((ACC_PER_TILE // HALF) * HALF, tail))
  return chunks


def _sc_mesh():
  return plsc.VectorSubcoreMesh(
      core_axis_name="c", subcore_axis_name="s", num_cores=NC, num_subcores=NS)


def _sc_sum(f_cat, src2, dstr, zeros_acc):
  IH = ROWS_PER_TILE // 2

  @functools.partial(
      pl.kernel,
      out_type=jax.ShapeDtypeStruct((NC, ACC_ROWS, HALF), jnp.float32),
      mesh=_sc_mesh(),
      scratch_types=[
          pltpu.VMEM_SHARED((ACC_ROWS, HALF), jnp.float32),
          pltpu.VMEM((IH, HALF), jnp.int32),
          pltpu.VMEM((IH, HALF), jnp.int32),
          pltpu.VMEM((HALF, HALF), jnp.float32),
          pltpu.VMEM((HALF, HALF), jnp.float32),
          pltpu.SemaphoreType.DMA,
          pltpu.SemaphoreType.DMA,
      ],
  )
  def k(f_hbm, src_hbm, dst_hbm, zacc_hbm, sum_out, acc, src_v, dst_v,
        rows_a, rows_b, sem_a, sem_b):
    cid = lax.axis_index("c")
    sid = lax.axis_index("s")
    a0 = sid * ACC_PER_TILE

    pltpu.sync_copy(zacc_hbm, rows_a)
    for off, n in _acc_chunks():
      pltpu.sync_copy(rows_a.at[pl.ds(0, n)], acc.at[pl.ds(a0 + off, n)])

    plsc.subcore_barrier()

    @pl.loop(0, 2)
    def _(h):
      r0 = sid * ROWS_PER_TILE + h * IH
      pltpu.sync_copy(src_hbm.at[cid, pl.ds(r0, IH)], src_v)
      pltpu.sync_copy(dst_hbm.at[pl.ds(r0, IH)], dst_v)

      def fire(b, rows, sem):
        pltpu.async_copy(f_hbm.at[src_v.at[b]], rows, sem)

      def drain(b, rows, sem):
        pltpu.make_async_copy(f_hbm.at[src_v.at[b]], rows, sem).wait()
        pltpu.sync_copy(rows, acc.at[dst_v.at[b]], add=True)

      fire(0, rows_a, sem_a)

      @pl.loop(0, IH // 2 - 1)
      def _(j):
        fire(2 * j + 1, rows_b, sem_b)
        drain(2 * j, rows_a, sem_a)
        fire(2 * j + 2, rows_a, sem_a)
        drain(2 * j + 1, rows_b, sem_b)

      fire(IH - 1, rows_b, sem_b)
      drain(IH - 2, rows_a, sem_a)
      drain(IH - 1, rows_b, sem_b)

    plsc.subcore_barrier()

    for off, n in _acc_chunks():
      pltpu.sync_copy(acc.at[pl.ds(a0 + off, n)], rows_a.at[pl.ds(0, n)])
      pltpu.sync_copy(rows_a.at[pl.ds(0, n)],
                      sum_out.at[cid, pl.ds(a0 + off, n)])

  return k(f_cat, src2, dstr, zeros_acc)


DEG_BATCHES = EPAD // NC // NS // HALF
DEG_ROWS = 10240
DEG_PER_TILE = DEG_ROWS // NS


def _sc_deg(dstr, zeros_rows, ones_rows):
  @functools.partial(
      pl.kernel,
      out_type=jax.ShapeDtypeStruct((NC, DEG_ROWS, HALF), jnp.float32),
      mesh=_sc_mesh(),
      scratch_types=[
          pltpu.VMEM_SHARED((DEG_ROWS, HALF), jnp.float32),
          pltpu.VMEM((DEG_BATCHES, HALF), jnp.int32),
          pltpu.VMEM((HALF, HALF), jnp.float32),
          pltpu.SemaphoreType.DMA,
      ],
  )
  def k(dst_hbm, zrows_hbm, ones_hbm, deg_out, deg, dst_v, ones_v, sem):
    cid = lax.axis_index("c")
    sid = lax.axis_index("s")
    a0 = sid * DEG_PER_TILE
    e0 = cid * NS * DEG_BATCHES + sid * DEG_BATCHES

    pltpu.sync_copy(zrows_hbm, ones_v)

    @pl.loop(0, DEG_PER_TILE // HALF)
    def _(i):
      pltpu.sync_copy(ones_v, deg.at[pl.ds(a0 + i * HALF, HALF)])

    pltpu.sync_copy(ones_hbm, ones_v)
    pltpu.sync_copy(dst_hbm.at[pl.ds(e0, DEG_BATCHES)], dst_v)

    plsc.subcore_barrier()

    @pl.loop(0, DEG_BATCHES)
    def _(j):
      pltpu.async_copy(ones_v, deg.at[dst_v.at[j]], add=True, sem=sem)

    @pl.loop(0, DEG_BATCHES)
    def _(j):
      pltpu.make_async_copy(ones_v, deg.at[dst_v.at[j]], sem).wait()

    plsc.subcore_barrier()

    @pl.loop(0, DEG_PER_TILE // HALF)
    def _(i):
      pltpu.sync_copy(deg.at[pl.ds(a0 + i * HALF, HALF)], ones_v)
      pltpu.sync_copy(ones_v, deg_out.at[cid, pl.ds(a0 + i * HALF, HALF)])

  return k(dstr, zeros_rows, ones_rows)


def _tc_mlp_body(feat_ref, lo_ref, hi_ref, dega_ref, degb_ref, w1_ref, b1_ref,
                 w2_ref, b2_ref, out_ref):
  deg = dega_ref[0][:, 0:1] + degb_ref[0][:, 0:1]
  dinv = 1.0 / jnp.maximum(deg, 1.0)
  lo = lo_ref[0] * dinv
  hi = hi_ref[0] * dinv
  x = feat_ref[...]
  pre = jnp.dot(x, w1_ref[0:FEAT], preferred_element_type=jnp.float32)
  pre += jnp.dot(lo, w1_ref[FEAT:FEAT + HALF], preferred_element_type=jnp.float32)
  pre += jnp.dot(hi, w1_ref[FEAT + HALF:2 * FEAT], preferred_element_type=jnp.float32)
  h = jnp.tanh(pre + b1_ref[...])
  out_ref[...] = (jnp.dot(h, w2_ref[...], preferred_element_type=jnp.float32)
                  + b2_ref[...])


def _tc_mlp(features, summed, deg16, W1, b1, W2, b2):
  R = 2000
  grid = (N_NODES // R,)
  return pl.pallas_call(
      _tc_mlp_body,
      grid=grid,
      in_specs=[
          pl.BlockSpec((R, FEAT), lambda i: (i, 0)),
          pl.BlockSpec((1, R, HALF), lambda i: (0, i, 0)),
          pl.BlockSpec((1, R, HALF), lambda i: (1, i, 0)),
          pl.BlockSpec((1, R, HALF), lambda i: (0, i, 0)),
          pl.BlockSpec((1, R, HALF), lambda i: (1, i, 0)),
          pl.BlockSpec((2 * FEAT, FEAT), lambda i: (0, 0)),
          pl.BlockSpec((1, FEAT), lambda i: (0, 0)),
          pl.BlockSpec((FEAT, FEAT), lambda i: (0, 0)),
          pl.BlockSpec((1, FEAT), lambda i: (0, 0)),
      ],
      out_specs=pl.BlockSpec((R, FEAT), lambda i: (i, 0)),
      out_shape=jax.ShapeDtypeStruct((N_NODES, FEAT), jnp.float32),
  )(features, summed, summed, deg16, deg16, W1, b1.reshape(1, FEAT), W2,
    b2.reshape(1, FEAT))


def kernel(nodes, features, edge_index, W1, b1, W2, b2):
  features = features.astype(jnp.float32)
  src = edge_index[0].astype(jnp.int32)
  dst = edge_index[1].astype(jnp.int32)

  f_cat = jnp.concatenate([features[:, :HALF], features[:, HALF:]], axis=0)

  pad = EPAD - N_EDGES
  srcp = jnp.concatenate([src, jnp.zeros((pad,), jnp.int32)])
  dstp = jnp.concatenate([dst, jnp.full((pad,), N_NODES, jnp.int32)])
  srcr = srcp.reshape(EPAD // HALF, HALF)
  src2 = jnp.stack([srcr, srcr + N_NODES])
  dstr = dstp.reshape(EPAD // HALF, HALF)

  zeros_acc = jnp.zeros((HALF, HALF), jnp.float32)
  ones_rows = jnp.ones((HALF, HALF), jnp.float32)

  summed = _sc_sum(f_cat, src2, dstr, zeros_acc)
  deg16 = _sc_deg(dstr, zeros_acc, ones_rows)
  return _tc_mlp(features, summed, deg16, W1, b1, W2, b2)
